# baseline (speedup 1.0000x reference)
import functools
import jax
import jax.numpy as jnp
from jax import lax
from jax.experimental import pallas as pl
from jax.experimental.pallas import tpu as pltpu
from jax.experimental.pallas import tpu_sc as plsc

D = 64
WIDE = 128
EPS = 1e-5

NC = 2
NS = 16
NW = NC * NS

CHUNK = 128


def _pad_body(t_ref, o_ref):
    o_ref[:, 0:D] = t_ref[...]


def _pad_table(table):
    v = table.shape[0]
    br = 8192
    return pl.pallas_call(
        _pad_body,
        grid=(v // br,),
        in_specs=[pl.BlockSpec((br, D), lambda i: (i, 0))],
        out_specs=pl.BlockSpec((br, WIDE), lambda i: (i, 0)),
        out_shape=jax.ShapeDtypeStruct((v, WIDE), jnp.float32),
    )(table)


def _make_sc_gather(n_rows):
    assert n_rows % (NW * CHUNK) == 0
    b_per_w = n_rows // NW
    n_chunks = b_per_w // CHUNK
    assert n_chunks % 2 == 0

    mesh = plsc.VectorSubcoreMesh(core_axis_name="c", subcore_axis_name="s")

    @functools.partial(
        pl.kernel,
        out_type=jax.ShapeDtypeStruct((n_rows, D), jnp.float32),
        mesh=mesh,
        compiler_params=pltpu.CompilerParams(use_tc_tiling_on_sc=False),
        scratch_types=[
            pltpu.VMEM((2, CHUNK), jnp.int32),
            pltpu.VMEM((2, CHUNK, D), jnp.float32),
            pltpu.SemaphoreType.DMA,
            pltpu.SemaphoreType.DMA,
        ],
    )
    def sc_gather(table_hbm, idx_hbm, out_hbm, idx_v, rows_v, gsem, osem):
        wid = lax.axis_index("s") * NC + lax.axis_index("c")
        base = wid * b_per_w

        pltpu.sync_copy(idx_hbm.at[pl.ds(base, CHUNK)], idx_v.at[0])
        pltpu.async_copy(table_hbm.at[idx_v.at[0]], rows_v.at[0], gsem)

        def pair_body(i, carry):
            for b in (0, 1):
                g = 2 * i + b
                @pl.when(g >= 1)
                def _(b=b, g=g):
                    pltpu.make_async_copy(
                        rows_v.at[1 - b],
                        out_hbm.at[pl.ds(base + (g - 1) * CHUNK, CHUNK)],
                        osem).wait()

                @pl.when(g + 1 < n_chunks)
                def _(b=b, g=g):
                    pltpu.sync_copy(
                        idx_hbm.at[pl.ds(base + (g + 1) * CHUNK, CHUNK)],
                        idx_v.at[1 - b])
                    pltpu.async_copy(table_hbm.at[idx_v.at[1 - b]],
                                     rows_v.at[1 - b], gsem)

                pltpu.make_async_copy(table_hbm.at[idx_v.at[b]],
                                      rows_v.at[b], gsem).wait()

                pltpu.async_copy(rows_v.at[b],
                                 out_hbm.at[pl.ds(base + g * CHUNK, CHUNK)],
                                 osem)
            return carry

        lax.fori_loop(0, n_chunks // 2, pair_body, 0, unroll=False)

        pltpu.make_async_copy(
            rows_v.at[1],
            out_hbm.at[pl.ds(base + (n_chunks - 1) * CHUNK, CHUNK)],
            osem).wait()

    return sc_gather


def _ln_body(e_ref, w_ref, b_ref, o_ref):
    e = e_ref[...]
    mean = jnp.mean(e, axis=-1, keepdims=True)
    var = jnp.mean(jnp.square(e - mean), axis=-1, keepdims=True)
    normed = (e - mean) / jnp.sqrt(var + EPS)
    o_ref[...] = normed * w_ref[...] + b_ref[...]


def _layernorm(e, w, b):
    n_rows = e.shape[0]
    br = 4096
    return pl.pallas_call(
        _ln_body,
        grid=(n_rows // br,),
        in_specs=[
            pl.BlockSpec((br, D), lambda i: (i, 0)),
            pl.BlockSpec((1, D), lambda i: (0, 0)),
            pl.BlockSpec((1, D), lambda i: (0, 0)),
        ],
        out_specs=pl.BlockSpec((br, D), lambda i: (i, 0)),
        out_shape=jax.ShapeDtypeStruct((n_rows, D), jnp.float32),
    )(e, w, b)


def kernel(x, table, ln_weight, ln_bias):
    batch, seq = x.shape
    n_rows = batch * seq
    xf = x.reshape(n_rows).astype(jnp.int32)
    e = _make_sc_gather(n_rows)(table, xf)
    out = _layernorm(e, ln_weight.reshape(1, D), ln_bias.reshape(1, D))
    return out.reshape(batch, seq, D)
